# baseline (device time: 10526 ns/iter reference)
import jax
import jax.numpy as jnp
from jax import lax
from jax.experimental import pallas as pl
from jax.experimental.pallas import tpu as pltpu

N_DEV = 4
N_TOK = 256
D_IN = 128
D_OUT = 256
E_LOCAL = 2
E_TOTAL = 8
ROWS = N_TOK // N_DEV


def kernel(x, router_W, route_idx, expert_W, shared_W):
    def body(x_ref, rw_ref, idx_ref, ew_ref, sw_ref, out_ref,
             send_buf, recv_buf, send_sems, recv_sems):
        my_pos = lax.axis_index("i")

        barrier_sem = pltpu.get_barrier_semaphore()
        for k in range(1, N_DEV):
            pl.semaphore_signal(
                barrier_sem, inc=1,
                device_id=((my_pos + k) % N_DEV,),
                device_id_type=pl.DeviceIdType.MESH,
            )
        pl.semaphore_wait(barrier_sem, N_DEV - 1)

        def expert_block(row0):
            xb = x_ref[pl.ds(row0, ROWS), :]
            scores = jnp.dot(xb, rw_ref[:, :],
                             preferred_element_type=jnp.float32)
            s_max = jnp.max(scores, axis=-1, keepdims=True)
            ex = jnp.exp(scores - s_max)
            probs = ex / jnp.sum(ex, axis=-1, keepdims=True)
            e_idx = idx_ref[pl.ds(row0, ROWS), :]
            iota = lax.broadcasted_iota(jnp.int32, (ROWS, E_TOTAL), 1)
            prob_sel = jnp.sum(jnp.where(iota == e_idx, probs, 0.0),
                               axis=-1, keepdims=True)
            acc = jnp.zeros((ROWS, D_OUT), jnp.float32)
            for j in range(E_LOCAL):
                cb = jnp.where(e_idx == E_LOCAL * my_pos + j, prob_sel, 0.0)
                acc = acc + cb * jnp.dot(xb, ew_ref[j, :, :],
                                         preferred_element_type=jnp.float32)
            return acc

        rdmas = []
        for k in (2, 1, 3):
            target = (my_pos + k) % N_DEV
            send_buf[k - 1, :, :] = expert_block(target * ROWS)
            rdma = pltpu.make_async_remote_copy(
                src_ref=send_buf.at[k - 1],
                dst_ref=recv_buf.at[k - 1],
                send_sem=send_sems.at[k - 1],
                recv_sem=recv_sems.at[k - 1],
                device_id=(target,),
                device_id_type=pl.DeviceIdType.MESH,
            )
            rdma.start()
            rdmas.append(rdma)

        x_blk = x_ref[pl.ds(my_pos * ROWS, ROWS), :]
        shared_blk = jnp.dot(x_blk, sw_ref[:, :],
                             preferred_element_type=jnp.float32)
        acc = shared_blk + expert_block(my_pos * ROWS)

        for rdma in rdmas:
            rdma.wait()
        acc = acc + recv_buf[0] + recv_buf[1] + recv_buf[2]
        out_ref[:, :] = acc

    return pl.pallas_call(
        body,
        out_shape=jax.ShapeDtypeStruct((ROWS, D_OUT), jnp.float32),
        in_specs=[
            pl.BlockSpec(memory_space=pltpu.VMEM),
            pl.BlockSpec(memory_space=pltpu.VMEM),
            pl.BlockSpec(memory_space=pltpu.VMEM),
            pl.BlockSpec(memory_space=pltpu.VMEM),
            pl.BlockSpec(memory_space=pltpu.VMEM),
        ],
        out_specs=pl.BlockSpec(memory_space=pltpu.VMEM),
        scratch_shapes=[
            pltpu.VMEM((N_DEV - 1, ROWS, D_OUT), jnp.float32),
            pltpu.VMEM((N_DEV - 1, ROWS, D_OUT), jnp.float32),
            pltpu.SemaphoreType.DMA((N_DEV - 1,)),
            pltpu.SemaphoreType.DMA((N_DEV - 1,)),
        ],
        compiler_params=pltpu.CompilerParams(collective_id=0),
    )(x, router_W, route_idx, expert_W, shared_W)


# device time: 9798 ns/iter; 1.0743x vs baseline; 1.0743x over previous
import jax
import jax.numpy as jnp
from jax import lax
from jax.experimental import pallas as pl
from jax.experimental.pallas import tpu as pltpu

N_DEV = 4
N_TOK = 256
D_IN = 128
D_OUT = 256
E_LOCAL = 2
E_TOTAL = 8
ROWS = N_TOK // N_DEV


def kernel(x, router_W, route_idx, expert_W, shared_W):
    def body(x_ref, rw_ref, idx_ref, ew_ref, sw_ref, out_ref,
             send_buf, recv_buf, send_sems, recv_sems):
        my_pos = lax.axis_index("i")

        barrier_sem = pltpu.get_barrier_semaphore()
        for k in range(1, N_DEV):
            pl.semaphore_signal(
                barrier_sem, inc=1,
                device_id=((my_pos + k) % N_DEV,),
                device_id_type=pl.DeviceIdType.MESH,
            )

        def expert_block(row0):
            xb = x_ref[pl.ds(row0, ROWS), :]
            scores = jnp.dot(xb, rw_ref[:, :],
                             preferred_element_type=jnp.float32)
            s_max = jnp.max(scores, axis=-1, keepdims=True)
            ex = jnp.exp(scores - s_max)
            probs = ex / jnp.sum(ex, axis=-1, keepdims=True)
            e_idx = idx_ref[pl.ds(row0, ROWS), :]
            iota = lax.broadcasted_iota(jnp.int32, (ROWS, E_TOTAL), 1)
            prob_sel = jnp.sum(jnp.where(iota == e_idx, probs, 0.0),
                               axis=-1, keepdims=True)
            acc = jnp.zeros((ROWS, D_OUT), jnp.float32)
            for j in range(E_LOCAL):
                cb = jnp.where(e_idx == E_LOCAL * my_pos + j, prob_sel, 0.0)
                acc = acc + cb * jnp.dot(xb, ew_ref[j, :, :],
                                         preferred_element_type=jnp.float32)
            return acc

        for k in (2, 1, 3):
            target = (my_pos + k) % N_DEV
            send_buf[k - 1, :, :] = expert_block(target * ROWS)

        pl.semaphore_wait(barrier_sem, N_DEV - 1)

        rdmas = []
        for k in (2, 1, 3):
            target = (my_pos + k) % N_DEV
            rdma = pltpu.make_async_remote_copy(
                src_ref=send_buf.at[k - 1],
                dst_ref=recv_buf.at[k - 1],
                send_sem=send_sems.at[k - 1],
                recv_sem=recv_sems.at[k - 1],
                device_id=(target,),
                device_id_type=pl.DeviceIdType.MESH,
            )
            rdma.start()
            rdmas.append(rdma)

        x_blk = x_ref[pl.ds(my_pos * ROWS, ROWS), :]
        shared_blk = jnp.dot(x_blk, sw_ref[:, :],
                             preferred_element_type=jnp.float32)
        acc = shared_blk + expert_block(my_pos * ROWS)

        for rdma in rdmas:
            rdma.wait()
        acc = acc + recv_buf[0] + recv_buf[1] + recv_buf[2]
        out_ref[:, :] = acc

    return pl.pallas_call(
        body,
        out_shape=jax.ShapeDtypeStruct((ROWS, D_OUT), jnp.float32),
        in_specs=[
            pl.BlockSpec(memory_space=pltpu.VMEM),
            pl.BlockSpec(memory_space=pltpu.VMEM),
            pl.BlockSpec(memory_space=pltpu.VMEM),
            pl.BlockSpec(memory_space=pltpu.VMEM),
            pl.BlockSpec(memory_space=pltpu.VMEM),
        ],
        out_specs=pl.BlockSpec(memory_space=pltpu.VMEM),
        scratch_shapes=[
            pltpu.VMEM((N_DEV - 1, ROWS, D_OUT), jnp.float32),
            pltpu.VMEM((N_DEV - 1, ROWS, D_OUT), jnp.float32),
            pltpu.SemaphoreType.DMA((N_DEV - 1,)),
            pltpu.SemaphoreType.DMA((N_DEV - 1,)),
        ],
        compiler_params=pltpu.CompilerParams(collective_id=0),
    )(x, router_W, route_idx, expert_W, shared_W)


# device time: 9164 ns/iter; 1.1486x vs baseline; 1.0692x over previous
import jax
import jax.numpy as jnp
from jax import lax
from jax.experimental import pallas as pl
from jax.experimental.pallas import tpu as pltpu

N_DEV = 4
N_TOK = 256
D_IN = 128
D_OUT = 256
E_LOCAL = 2
E_TOTAL = 8
ROWS = N_TOK // N_DEV


def kernel(x, router_W, route_idx, expert_W, shared_W):
    def body(x_ref, rw_ref, idx_ref, ew_ref, sw_ref, out_ref,
             send_buf, recv_buf, send_sems, recv_sems):
        my_pos = lax.axis_index("i")

        barrier_sem = pltpu.get_barrier_semaphore()
        for k in range(1, N_DEV):
            pl.semaphore_signal(
                barrier_sem, inc=1,
                device_id=((my_pos + k) % N_DEV,),
                device_id_type=pl.DeviceIdType.MESH,
            )

        def expert_block(row0):
            xb = x_ref[pl.ds(row0, ROWS), :]
            scores = jnp.dot(xb, rw_ref[:, :],
                             preferred_element_type=jnp.float32)
            s_max = jnp.max(scores, axis=-1, keepdims=True)
            ex = jnp.exp(scores - s_max)
            probs = ex / jnp.sum(ex, axis=-1, keepdims=True)
            e_idx = idx_ref[pl.ds(row0, ROWS), :]
            iota = lax.broadcasted_iota(jnp.int32, (ROWS, E_TOTAL), 1)
            prob_sel = jnp.sum(jnp.where(iota == e_idx, probs, 0.0),
                               axis=-1, keepdims=True)
            acc = jnp.zeros((ROWS, D_OUT), jnp.float32)
            for j in range(E_LOCAL):
                cb = jnp.where(e_idx == E_LOCAL * my_pos + j, prob_sel, 0.0)
                acc = acc + cb * jnp.dot(xb, ew_ref[j, :, :],
                                         preferred_element_type=jnp.float32)
            return acc

        for k in (2, 1, 3):
            target = (my_pos + k) % N_DEV
            send_buf[k - 1, :, :] = expert_block(target * ROWS).astype(
                jnp.bfloat16)

        pl.semaphore_wait(barrier_sem, N_DEV - 1)

        rdmas = []
        for k in (2, 1, 3):
            target = (my_pos + k) % N_DEV
            rdma = pltpu.make_async_remote_copy(
                src_ref=send_buf.at[k - 1],
                dst_ref=recv_buf.at[k - 1],
                send_sem=send_sems.at[k - 1],
                recv_sem=recv_sems.at[k - 1],
                device_id=(target,),
                device_id_type=pl.DeviceIdType.MESH,
            )
            rdma.start()
            rdmas.append(rdma)

        x_blk = x_ref[pl.ds(my_pos * ROWS, ROWS), :]
        shared_blk = jnp.dot(x_blk, sw_ref[:, :],
                             preferred_element_type=jnp.float32)
        acc = shared_blk + expert_block(my_pos * ROWS)

        for rdma in rdmas:
            rdma.wait()
        for s in range(N_DEV - 1):
            acc = acc + recv_buf[s].astype(jnp.float32)
        out_ref[:, :] = acc

    return pl.pallas_call(
        body,
        out_shape=jax.ShapeDtypeStruct((ROWS, D_OUT), jnp.float32),
        in_specs=[
            pl.BlockSpec(memory_space=pltpu.VMEM),
            pl.BlockSpec(memory_space=pltpu.VMEM),
            pl.BlockSpec(memory_space=pltpu.VMEM),
            pl.BlockSpec(memory_space=pltpu.VMEM),
            pl.BlockSpec(memory_space=pltpu.VMEM),
        ],
        out_specs=pl.BlockSpec(memory_space=pltpu.VMEM),
        scratch_shapes=[
            pltpu.VMEM((N_DEV - 1, ROWS, D_OUT), jnp.bfloat16),
            pltpu.VMEM((N_DEV - 1, ROWS, D_OUT), jnp.bfloat16),
            pltpu.SemaphoreType.DMA((N_DEV - 1,)),
            pltpu.SemaphoreType.DMA((N_DEV - 1,)),
        ],
        compiler_params=pltpu.CompilerParams(collective_id=0),
    )(x, router_W, route_idx, expert_W, shared_W)


# device time: 9123 ns/iter; 1.1538x vs baseline; 1.0045x over previous
import jax
import jax.numpy as jnp
from jax import lax
from jax.experimental import pallas as pl
from jax.experimental.pallas import tpu as pltpu

N_DEV = 4
N_TOK = 256
D_IN = 128
D_OUT = 256
E_LOCAL = 2
E_TOTAL = 8
ROWS = N_TOK // N_DEV


def kernel(x, router_W, route_idx, expert_W, shared_W):
    def body(x_ref, rw_ref, idx_ref, ew_ref, sw_ref, out_ref,
             partial_ref, recv_buf, send_sems, recv_sems):
        my_pos = lax.axis_index("i")

        barrier_sem = pltpu.get_barrier_semaphore()
        for k in range(1, N_DEV):
            pl.semaphore_signal(
                barrier_sem, inc=1,
                device_id=((my_pos + k) % N_DEV,),
                device_id_type=pl.DeviceIdType.MESH,
            )

        xv = x_ref[:, :]
        scores = jnp.dot(xv, rw_ref[:, :], preferred_element_type=jnp.float32)
        s_max = jnp.max(scores, axis=-1, keepdims=True)
        ex = jnp.exp(scores - s_max)
        probs = ex / jnp.sum(ex, axis=-1, keepdims=True)

        e_idx = idx_ref[:, :]
        iota = lax.broadcasted_iota(jnp.int32, (N_TOK, E_TOTAL), 1)
        prob_sel = jnp.sum(jnp.where(iota == e_idx, probs, 0.0), axis=-1,
                           keepdims=True)

        contrib = jnp.zeros((N_TOK, D_OUT), jnp.float32)
        for j in range(E_LOCAL):
            coef = jnp.where(e_idx == E_LOCAL * my_pos + j, prob_sel, 0.0)
            contrib = contrib + coef * jnp.dot(
                xv, ew_ref[j, :, :], preferred_element_type=jnp.float32)
        partial_ref[:, :] = contrib.astype(jnp.bfloat16)

        pl.semaphore_wait(barrier_sem, N_DEV - 1)

        rdmas = []
        for k in (2, 1, 3):
            target = (my_pos + k) % N_DEV
            rdma = pltpu.make_async_remote_copy(
                src_ref=partial_ref.at[pl.ds(target * ROWS, ROWS), :],
                dst_ref=recv_buf.at[k - 1],
                send_sem=send_sems.at[k - 1],
                recv_sem=recv_sems.at[k - 1],
                device_id=(target,),
                device_id_type=pl.DeviceIdType.MESH,
            )
            rdma.start()
            rdmas.append((k - 1, rdma))

        x_blk = x_ref[pl.ds(my_pos * ROWS, ROWS), :]
        shared_blk = jnp.dot(x_blk, sw_ref[:, :],
                             preferred_element_type=jnp.float32)
        acc = shared_blk + partial_ref[
            pl.ds(my_pos * ROWS, ROWS), :].astype(jnp.float32)

        for slot, rdma in rdmas:
            rdma.wait()
            acc = acc + recv_buf[slot].astype(jnp.float32)
        out_ref[:, :] = acc

    return pl.pallas_call(
        body,
        out_shape=jax.ShapeDtypeStruct((ROWS, D_OUT), jnp.float32),
        in_specs=[
            pl.BlockSpec(memory_space=pltpu.VMEM),
            pl.BlockSpec(memory_space=pltpu.VMEM),
            pl.BlockSpec(memory_space=pltpu.VMEM),
            pl.BlockSpec(memory_space=pltpu.VMEM),
            pl.BlockSpec(memory_space=pltpu.VMEM),
        ],
        out_specs=pl.BlockSpec(memory_space=pltpu.VMEM),
        scratch_shapes=[
            pltpu.VMEM((N_TOK, D_OUT), jnp.bfloat16),
            pltpu.VMEM((N_DEV - 1, ROWS, D_OUT), jnp.bfloat16),
            pltpu.SemaphoreType.DMA((N_DEV - 1,)),
            pltpu.SemaphoreType.DMA((N_DEV - 1,)),
        ],
        compiler_params=pltpu.CompilerParams(collective_id=0),
    )(x, router_W, route_idx, expert_W, shared_W)
